# Initial kernel scaffold; baseline (speedup 1.0000x reference)
import jax
import jax.numpy as jnp
from jax.experimental import pallas as pl


def kernel(seeds, nbr1, nbr2, emb, W1s, W1n, W2s, W2n):
    raise NotImplementedError("write your pallas kernel here")


# speedup vs baseline: 2.7682x; 2.7682x over previous
import functools

import jax
import jax.numpy as jnp
from jax import lax
from jax.experimental import pallas as pl
from jax.experimental.pallas import tpu as pltpu
from jax.experimental.pallas import tpu_sc as plsc

_NW = 32
_CH = 128
_SCH = 64


def _sc_gather(emb, nbr2f, sel):
    V, D = emb.shape
    N2 = nbr2f.shape[0]
    NS = sel.shape[0]
    n2_w = N2 // _NW
    n2_ch = n2_w // _CH
    sel_w = NS // _NW
    sel_ch = sel_w // _SCH

    mesh = plsc.VectorSubcoreMesh(core_axis_name="c", subcore_axis_name="s")

    @functools.partial(
        pl.kernel,
        mesh=mesh,
        out_type=[
            jax.ShapeDtypeStruct((N2, D), emb.dtype),
            jax.ShapeDtypeStruct((NS, D), emb.dtype),
        ],
        scratch_types=[
            pltpu.VMEM((_CH,), jnp.int32),
            pltpu.VMEM((_CH, D), emb.dtype),
            pltpu.VMEM((_SCH,), jnp.int32),
            pltpu.VMEM((_SCH, D), emb.dtype),
            pltpu.SemaphoreType.DMA,
        ],
    )
    def k(emb_hbm, n2_hbm, sel_hbm, h2_hbm, hs_hbm,
          idx_v, rows_v, sidx_v, srows_v, sem):
        wid = lax.axis_index("s") * 2 + lax.axis_index("c")

        base2 = wid * n2_w

        @pl.loop(0, n2_ch)
        def _(c):
            off = base2 + c * _CH
            pltpu.sync_copy(n2_hbm.at[pl.ds(off, _CH)], idx_v)
            pltpu.async_copy(emb_hbm.at[idx_v], rows_v, sem).wait()
            pltpu.sync_copy(rows_v, h2_hbm.at[pl.ds(off, _CH)])

        bases = wid * sel_w

        @pl.loop(0, sel_ch)
        def _(c):
            off = bases + c * _SCH
            pltpu.sync_copy(sel_hbm.at[pl.ds(off, _SCH)], sidx_v)
            pltpu.async_copy(emb_hbm.at[sidx_v], srows_v, sem).wait()
            pltpu.sync_copy(srows_v, hs_hbm.at[pl.ds(off, _SCH)])

    return k(emb, nbr2f, sel)


def _tc_dense(h2v, hsel, G1T, W1s, W1n, W2s, W2n, B, F1, F2, D, H):
    BLK = 1024
    OB = BLK // F1
    nblk = (B * F1) // BLK

    def body(h2v_ref, hs_ref, h0_ref, g_ref, w1s_ref, w1n_ref, w2s_ref,
             w2n_ref, o_ref):
        h2m = h2v_ref[:, 0:D]
        for kk in range(1, F2):
            h2m = h2m + h2v_ref[:, kk * D:(kk + 1) * D]
        h2m = h2m * (1.0 / F2)
        h1 = jnp.maximum(
            jnp.dot(hs_ref[...], w1s_ref[...],
                    preferred_element_type=jnp.float32)
            + jnp.dot(h2m, w1n_ref[...], preferred_element_type=jnp.float32),
            0.0)
        h1m = jnp.dot(g_ref[...], h1, preferred_element_type=jnp.float32)
        o_ref[...] = jnp.maximum(
            jnp.dot(h0_ref[...], w2s_ref[...],
                    preferred_element_type=jnp.float32)
            + jnp.dot(h1m, w2n_ref[...], preferred_element_type=jnp.float32),
            0.0)

    return pl.pallas_call(
        body,
        grid=(nblk,),
        in_specs=[
            pl.BlockSpec((BLK, F2 * D), lambda i: (i, 0)),
            pl.BlockSpec((BLK, D), lambda i: (i, 0)),
            pl.BlockSpec((OB, D), lambda i: (i + (B * F1) // OB, 0)),
            pl.BlockSpec((OB, BLK), lambda i: (0, 0)),
            pl.BlockSpec((D, H), lambda i: (0, 0)),
            pl.BlockSpec((D, H), lambda i: (0, 0)),
            pl.BlockSpec((D, H), lambda i: (0, 0)),
            pl.BlockSpec((H, H), lambda i: (0, 0)),
        ],
        out_specs=pl.BlockSpec((OB, H), lambda i: (i, 0)),
        out_shape=jax.ShapeDtypeStruct((B, H), jnp.float32),
    )(h2v, hsel, hsel, G1T, W1s, W1n, W2s, W2n)


def kernel(seeds, nbr1, nbr2, emb, W1s, W1n, W2s, W2n):
    B, F1 = nbr1.shape
    F2 = nbr2.shape[1]
    D = emb.shape[1]
    H = W1s.shape[1]

    nbr2f = nbr2.reshape(-1)
    sel = jnp.concatenate([nbr1.reshape(-1), seeds])

    h2, hsel = _sc_gather(emb, nbr2f, sel)
    h2v = h2.reshape(B * F1, F2 * D)

    BLK = 1024
    OB = BLK // F1
    G1T = (jnp.arange(OB, dtype=jnp.int32)[:, None]
           == (jnp.arange(BLK, dtype=jnp.int32)[None, :] // F1)
           ).astype(jnp.float32) * (1.0 / F1)

    return _tc_dense(h2v, hsel, G1T, W1s, W1n, W2s, W2n, B, F1, F2, D, H)


# speedup vs baseline: 4.1970x; 1.5161x over previous
import functools

import jax
import jax.numpy as jnp
from jax import lax
from jax.experimental import pallas as pl
from jax.experimental.pallas import tpu as pltpu
from jax.experimental.pallas import tpu_sc as plsc

_NW = 32
_CH = 128
_SCH = 64


def _sc_gather(emb, nbr2f, sel):
    V, D = emb.shape
    N2 = nbr2f.shape[0]
    NS = sel.shape[0]
    n2_w = N2 // _NW
    n2_ch = n2_w // _CH
    sel_w = NS // _NW
    sel_ch = sel_w // _SCH

    mesh = plsc.VectorSubcoreMesh(core_axis_name="c", subcore_axis_name="s")

    @functools.partial(
        pl.kernel,
        mesh=mesh,
        out_type=[
            jax.ShapeDtypeStruct((N2, D), emb.dtype),
            jax.ShapeDtypeStruct((NS, D), emb.dtype),
        ],
        scratch_types=[
            pltpu.VMEM((_CH,), jnp.int32),
            pltpu.VMEM((_CH, D), emb.dtype),
            pltpu.VMEM((_SCH,), jnp.int32),
            pltpu.VMEM((_SCH, D), emb.dtype),
            pltpu.SemaphoreType.DMA,
        ],
    )
    def k(emb_hbm, n2_hbm, sel_hbm, h2_hbm, hs_hbm,
          idx_v, rows_v, sidx_v, srows_v, sem):
        wid = lax.axis_index("s") * 2 + lax.axis_index("c")

        base2 = wid * n2_w

        @pl.loop(0, n2_ch)
        def _(c):
            off = base2 + c * _CH
            pltpu.sync_copy(n2_hbm.at[pl.ds(off, _CH)], idx_v)
            pltpu.async_copy(emb_hbm.at[idx_v], rows_v, sem).wait()
            pltpu.sync_copy(rows_v, h2_hbm.at[pl.ds(off, _CH)])

        bases = wid * sel_w

        @pl.loop(0, sel_ch)
        def _(c):
            off = bases + c * _SCH
            pltpu.sync_copy(sel_hbm.at[pl.ds(off, _SCH)], sidx_v)
            pltpu.async_copy(emb_hbm.at[sidx_v], srows_v, sem).wait()
            pltpu.sync_copy(srows_v, hs_hbm.at[pl.ds(off, _SCH)])

    return k(emb, nbr2f, sel)


def _tc_dense(h2v, hsel, G1T, W1s, W1n, W2s, W2n, B, F1, F2, D, H):
    BLK = 1024
    OB = BLK // F1
    nblk = (B * F1) // BLK

    def body(h2v_ref, hs_ref, h0_ref, g_ref, w1s_ref, w1n_ref, w2s_ref,
             w2n_ref, o_ref):
        h2m = jnp.sum(h2v_ref[...].reshape(BLK, F2, D), axis=1)
        h2m = h2m * (1.0 / F2)
        h1 = jnp.maximum(
            jnp.dot(hs_ref[...], w1s_ref[...],
                    preferred_element_type=jnp.float32)
            + jnp.dot(h2m, w1n_ref[...], preferred_element_type=jnp.float32),
            0.0)
        h1m = jnp.dot(g_ref[...], h1, preferred_element_type=jnp.float32)
        o_ref[...] = jnp.maximum(
            jnp.dot(h0_ref[...], w2s_ref[...],
                    preferred_element_type=jnp.float32)
            + jnp.dot(h1m, w2n_ref[...], preferred_element_type=jnp.float32),
            0.0)

    return pl.pallas_call(
        body,
        grid=(nblk,),
        in_specs=[
            pl.BlockSpec((BLK * F2, D), lambda i: (i, 0)),
            pl.BlockSpec((BLK, D), lambda i: (i, 0)),
            pl.BlockSpec((OB, D), lambda i: (i + (B * F1) // OB, 0)),
            pl.BlockSpec((OB, BLK), lambda i: (0, 0)),
            pl.BlockSpec((D, H), lambda i: (0, 0)),
            pl.BlockSpec((D, H), lambda i: (0, 0)),
            pl.BlockSpec((D, H), lambda i: (0, 0)),
            pl.BlockSpec((H, H), lambda i: (0, 0)),
        ],
        out_specs=pl.BlockSpec((OB, H), lambda i: (i, 0)),
        out_shape=jax.ShapeDtypeStruct((B, H), jnp.float32),
    )(h2v, hsel, hsel, G1T, W1s, W1n, W2s, W2n)


def kernel(seeds, nbr1, nbr2, emb, W1s, W1n, W2s, W2n):
    B, F1 = nbr1.shape
    F2 = nbr2.shape[1]
    D = emb.shape[1]
    H = W1s.shape[1]

    nbr2f = nbr2.reshape(-1)
    sel = jnp.concatenate([nbr1.reshape(-1), seeds])

    h2, hsel = _sc_gather(emb, nbr2f, sel)

    BLK = 1024
    OB = BLK // F1
    G1T = (jnp.arange(OB, dtype=jnp.int32)[:, None]
           == (jnp.arange(BLK, dtype=jnp.int32)[None, :] // F1)
           ).astype(jnp.float32) * (1.0 / F1)

    return _tc_dense(h2, hsel, G1T, W1s, W1n, W2s, W2n, B, F1, F2, D, H)


# speedup vs baseline: 6.0739x; 1.4472x over previous
import functools

import jax
import jax.numpy as jnp
from jax import lax
from jax.experimental import pallas as pl
from jax.experimental.pallas import tpu as pltpu
from jax.experimental.pallas import tpu_sc as plsc

_NW = 32
_CH = 128
_SCH = 64
_NB = 4


def _sc_gather(emb, nbr2f, sel):
    V, D = emb.shape
    N2 = nbr2f.shape[0]
    NS = sel.shape[0]
    n2_w = N2 // _NW
    n2_ch = n2_w // _CH
    sel_w = NS // _NW
    sel_ch = sel_w // _SCH

    mesh = plsc.VectorSubcoreMesh(core_axis_name="c", subcore_axis_name="s")

    @functools.partial(
        pl.kernel,
        mesh=mesh,
        out_type=[
            jax.ShapeDtypeStruct((N2, D), emb.dtype),
            jax.ShapeDtypeStruct((NS, D), emb.dtype),
        ],
        scratch_types=[
            pltpu.VMEM((_NB, _CH), jnp.int32),
            pltpu.VMEM((_NB, _CH, D), emb.dtype),
            pltpu.VMEM((_SCH,), jnp.int32),
            pltpu.VMEM((_SCH, D), emb.dtype),
            pltpu.SemaphoreType.DMA,
            pltpu.SemaphoreType.DMA((_NB,)),
            pltpu.SemaphoreType.DMA((_NB,)),
        ],
    )
    def k(emb_hbm, n2_hbm, sel_hbm, h2_hbm, hs_hbm,
          idx_v, rows_v, sidx_v, srows_v, sem, gs, ws):
        wid = lax.axis_index("s") * 2 + lax.axis_index("c")

        base2 = wid * n2_w

        @pl.loop(0, n2_ch, step=_NB)
        def _(c):
            for b in range(_NB):
                cc = c + b
                off = base2 + cc * _CH

                @pl.when(cc >= _NB)
                def _():
                    pltpu.make_async_copy(
                        rows_v.at[b], h2_hbm.at[pl.ds(0, _CH)], ws.at[b]
                    ).wait()

                pltpu.sync_copy(n2_hbm.at[pl.ds(off, _CH)], idx_v.at[b])
                pltpu.async_copy(emb_hbm.at[idx_v.at[b]], rows_v.at[b],
                                 gs.at[b])
                pb = (b - 1) % _NB

                @pl.when(cc >= 1)
                def _():
                    pltpu.make_async_copy(
                        emb_hbm.at[idx_v.at[pb]], rows_v.at[pb], gs.at[pb]
                    ).wait()
                    pltpu.async_copy(rows_v.at[pb],
                                     h2_hbm.at[pl.ds(off - _CH, _CH)],
                                     ws.at[pb])

        lb = (n2_ch - 1) % _NB
        pltpu.make_async_copy(emb_hbm.at[idx_v.at[lb]], rows_v.at[lb],
                              gs.at[lb]).wait()
        pltpu.async_copy(rows_v.at[lb],
                         h2_hbm.at[pl.ds(base2 + (n2_ch - 1) * _CH, _CH)],
                         ws.at[lb])
        for b in range(_NB):
            pltpu.make_async_copy(rows_v.at[b], h2_hbm.at[pl.ds(0, _CH)],
                                  ws.at[b]).wait()

        bases = wid * sel_w

        @pl.loop(0, sel_ch)
        def _(c):
            off = bases + c * _SCH
            pltpu.sync_copy(sel_hbm.at[pl.ds(off, _SCH)], sidx_v)
            pltpu.async_copy(emb_hbm.at[sidx_v], srows_v, sem).wait()
            pltpu.sync_copy(srows_v, hs_hbm.at[pl.ds(off, _SCH)])

    return k(emb, nbr2f, sel)


def _tc_dense(h2v, hsel, G1T, W1s, W1n, W2s, W2n, B, F1, F2, D, H):
    BLK = 1024
    OB = BLK // F1
    nblk = (B * F1) // BLK

    def body(h2v_ref, hs_ref, h0_ref, g_ref, w1s_ref, w1n_ref, w2s_ref,
             w2n_ref, o_ref):
        h2m = jnp.sum(h2v_ref[...].reshape(BLK, F2, D), axis=1)
        h2m = h2m * (1.0 / F2)
        h1 = jnp.maximum(
            jnp.dot(hs_ref[...], w1s_ref[...],
                    preferred_element_type=jnp.float32)
            + jnp.dot(h2m, w1n_ref[...], preferred_element_type=jnp.float32),
            0.0)
        h1m = jnp.dot(g_ref[...], h1, preferred_element_type=jnp.float32)
        o_ref[...] = jnp.maximum(
            jnp.dot(h0_ref[...], w2s_ref[...],
                    preferred_element_type=jnp.float32)
            + jnp.dot(h1m, w2n_ref[...], preferred_element_type=jnp.float32),
            0.0)

    return pl.pallas_call(
        body,
        grid=(nblk,),
        in_specs=[
            pl.BlockSpec((BLK * F2, D), lambda i: (i, 0)),
            pl.BlockSpec((BLK, D), lambda i: (i, 0)),
            pl.BlockSpec((OB, D), lambda i: (i + (B * F1) // OB, 0)),
            pl.BlockSpec((OB, BLK), lambda i: (0, 0)),
            pl.BlockSpec((D, H), lambda i: (0, 0)),
            pl.BlockSpec((D, H), lambda i: (0, 0)),
            pl.BlockSpec((D, H), lambda i: (0, 0)),
            pl.BlockSpec((H, H), lambda i: (0, 0)),
        ],
        out_specs=pl.BlockSpec((OB, H), lambda i: (i, 0)),
        out_shape=jax.ShapeDtypeStruct((B, H), jnp.float32),
    )(h2v, hsel, hsel, G1T, W1s, W1n, W2s, W2n)


def kernel(seeds, nbr1, nbr2, emb, W1s, W1n, W2s, W2n):
    B, F1 = nbr1.shape
    F2 = nbr2.shape[1]
    D = emb.shape[1]
    H = W1s.shape[1]

    nbr2f = nbr2.reshape(-1)
    sel = jnp.concatenate([nbr1.reshape(-1), seeds])

    h2, hsel = _sc_gather(emb, nbr2f, sel)

    BLK = 1024
    OB = BLK // F1
    G1T = (jnp.arange(OB, dtype=jnp.int32)[:, None]
           == (jnp.arange(BLK, dtype=jnp.int32)[None, :] // F1)
           ).astype(jnp.float32) * (1.0 / F1)

    return _tc_dense(h2, hsel, G1T, W1s, W1n, W2s, W2n, B, F1, F2, D, H)


# speedup vs baseline: 6.0762x; 1.0004x over previous
import functools

import jax
import jax.numpy as jnp
from jax import lax
from jax.experimental import pallas as pl
from jax.experimental.pallas import tpu as pltpu
from jax.experimental.pallas import tpu_sc as plsc

_NW = 32
_CH = 128
_SCH = 64
_NB = 4


def _sc_gather(emb, emb16, nbr2f, sel):
    V, D = emb.shape
    N2 = nbr2f.shape[0]
    NS = sel.shape[0]
    n2_w = N2 // _NW
    n2_ch = n2_w // _CH
    sel_w = NS // _NW
    sel_ch = sel_w // _SCH

    mesh = plsc.VectorSubcoreMesh(core_axis_name="c", subcore_axis_name="s")

    @functools.partial(
        pl.kernel,
        mesh=mesh,
        out_type=[
            jax.ShapeDtypeStruct((N2, D), emb16.dtype),
            jax.ShapeDtypeStruct((NS, D), emb.dtype),
        ],
        scratch_types=[
            pltpu.VMEM((_NB, _CH), jnp.int32),
            pltpu.VMEM((_NB, _CH, D), emb16.dtype),
            pltpu.VMEM((_SCH,), jnp.int32),
            pltpu.VMEM((_SCH, D), emb.dtype),
            pltpu.SemaphoreType.DMA,
            pltpu.SemaphoreType.DMA((_NB,)),
            pltpu.SemaphoreType.DMA((_NB,)),
        ],
    )
    def k(emb_hbm, emb16_hbm, n2_hbm, sel_hbm, h2_hbm, hs_hbm,
          idx_v, rows_v, sidx_v, srows_v, sem, gs, ws):
        wid = lax.axis_index("s") * 2 + lax.axis_index("c")

        base2 = wid * n2_w

        @pl.loop(0, n2_ch, step=_NB)
        def _(c):
            for b in range(_NB):
                cc = c + b
                off = base2 + cc * _CH

                @pl.when(cc >= _NB)
                def _():
                    pltpu.make_async_copy(
                        rows_v.at[b], h2_hbm.at[pl.ds(0, _CH)], ws.at[b]
                    ).wait()

                pltpu.sync_copy(n2_hbm.at[pl.ds(off, _CH)], idx_v.at[b])
                pltpu.async_copy(emb16_hbm.at[idx_v.at[b]], rows_v.at[b],
                                 gs.at[b])
                pb = (b - 1) % _NB

                @pl.when(cc >= 1)
                def _():
                    pltpu.make_async_copy(
                        emb16_hbm.at[idx_v.at[pb]], rows_v.at[pb], gs.at[pb]
                    ).wait()
                    pltpu.async_copy(rows_v.at[pb],
                                     h2_hbm.at[pl.ds(off - _CH, _CH)],
                                     ws.at[pb])

        lb = (n2_ch - 1) % _NB
        pltpu.make_async_copy(emb16_hbm.at[idx_v.at[lb]], rows_v.at[lb],
                              gs.at[lb]).wait()
        pltpu.async_copy(rows_v.at[lb],
                         h2_hbm.at[pl.ds(base2 + (n2_ch - 1) * _CH, _CH)],
                         ws.at[lb])
        for b in range(_NB):
            pltpu.make_async_copy(rows_v.at[b], h2_hbm.at[pl.ds(0, _CH)],
                                  ws.at[b]).wait()

        bases = wid * sel_w

        @pl.loop(0, sel_ch)
        def _(c):
            off = bases + c * _SCH
            pltpu.sync_copy(sel_hbm.at[pl.ds(off, _SCH)], sidx_v)
            pltpu.async_copy(emb_hbm.at[sidx_v], srows_v, sem).wait()
            pltpu.sync_copy(srows_v, hs_hbm.at[pl.ds(off, _SCH)])

    return k(emb, emb16, nbr2f, sel)


def _tc_dense(h2v, hsel, G1T, W1s, W1n, W2s, W2n, B, F1, F2, D, H):
    BLK = 1024
    OB = BLK // F1
    nblk = (B * F1) // BLK

    def body(h2v_ref, hs_ref, h0_ref, g_ref, w1s_ref, w1n_ref, w2s_ref,
             w2n_ref, o_ref):
        h2m = jnp.sum(h2v_ref[...].astype(jnp.float32).reshape(BLK, F2, D),
                      axis=1)
        h2m = h2m * (1.0 / F2)
        h1 = jnp.maximum(
            jnp.dot(hs_ref[...], w1s_ref[...],
                    preferred_element_type=jnp.float32)
            + jnp.dot(h2m, w1n_ref[...], preferred_element_type=jnp.float32),
            0.0)
        h1m = jnp.dot(g_ref[...], h1, preferred_element_type=jnp.float32)
        o_ref[...] = jnp.maximum(
            jnp.dot(h0_ref[...], w2s_ref[...],
                    preferred_element_type=jnp.float32)
            + jnp.dot(h1m, w2n_ref[...], preferred_element_type=jnp.float32),
            0.0)

    return pl.pallas_call(
        body,
        grid=(nblk,),
        in_specs=[
            pl.BlockSpec((BLK * F2, D), lambda i: (i, 0)),
            pl.BlockSpec((BLK, D), lambda i: (i, 0)),
            pl.BlockSpec((OB, D), lambda i: (i + (B * F1) // OB, 0)),
            pl.BlockSpec((OB, BLK), lambda i: (0, 0)),
            pl.BlockSpec((D, H), lambda i: (0, 0)),
            pl.BlockSpec((D, H), lambda i: (0, 0)),
            pl.BlockSpec((D, H), lambda i: (0, 0)),
            pl.BlockSpec((H, H), lambda i: (0, 0)),
        ],
        out_specs=pl.BlockSpec((OB, H), lambda i: (i, 0)),
        out_shape=jax.ShapeDtypeStruct((B, H), jnp.float32),
    )(h2v, hsel, hsel, G1T, W1s, W1n, W2s, W2n)


def kernel(seeds, nbr1, nbr2, emb, W1s, W1n, W2s, W2n):
    B, F1 = nbr1.shape
    F2 = nbr2.shape[1]
    D = emb.shape[1]
    H = W1s.shape[1]

    nbr2f = nbr2.reshape(-1)
    sel = jnp.concatenate([nbr1.reshape(-1), seeds])

    h2, hsel = _sc_gather(emb, emb.astype(jnp.float32), nbr2f, sel)

    BLK = 1024
    OB = BLK // F1
    G1T = (jnp.arange(OB, dtype=jnp.int32)[:, None]
           == (jnp.arange(BLK, dtype=jnp.int32)[None, :] // F1)
           ).astype(jnp.float32) * (1.0 / F1)

    return _tc_dense(h2, hsel, G1T, W1s, W1n, W2s, W2n, B, F1, F2, D, H)
